# Initial kernel scaffold; baseline (speedup 1.0000x reference)
import jax
import jax.numpy as jnp
from jax.experimental import pallas as pl


def kernel(x):
    raise NotImplementedError("write your pallas kernel here")


# speedup vs baseline: 10.1492x; 10.1492x over previous
import functools

import jax
import jax.numpy as jnp
from jax.experimental import pallas as pl
from jax.experimental.pallas import tpu as pltpu

_GO = -11.0
_GE = -1.0
_NEG = -1e30
_A = 1023
_N = 2 * _A - 1
_KP = 1024


def _skew(mat):
    a, b = mat.shape
    n = a + b - 1
    p = jnp.pad(mat, ((0, 0), (0, n + 1 - b)), constant_values=_NEG)
    return p.reshape(-1)[: a * n].reshape(a, n)


def _shift_k(u):
    return jnp.concatenate(
        [jnp.full(u.shape[:-1] + (1,), _NEG, u.dtype), u[..., :-1]], axis=-1)


def _sw_kernel(y_ref, z_ref, out_ref, g2_ref, g1_ref, m_ref, s_ref):
    t = pl.program_id(0)

    @pl.when(t == 0)
    def _():
        g2_ref[...] = jnp.full((3, _KP), _NEG, jnp.float32)
        g1_ref[...] = jnp.full((3, _KP), _NEG, jnp.float32)
        m_ref[...] = jnp.full((1, _KP), _NEG, jnp.float32)
        s_ref[...] = jnp.zeros((1, _KP), jnp.float32)

    y = y_ref[0, 0, :]
    z = z_ref[0]
    g2 = g2_ref[...]
    g1 = g1_ref[...]

    g2s = _shift_k(g2)
    m0 = jnp.maximum(jnp.maximum(g2s[0], g2s[1]),
                     jnp.maximum(g2s[2], 0.0))
    aln = y + m0 + jnp.log(
        jnp.exp(g2s[0] - m0) + jnp.exp(g2s[1] - m0)
        + jnp.exp(g2s[2] - m0) + jnp.exp(-m0))

    r0 = g1[0] + _GO
    r1 = g1[1] + _GE
    mr = jnp.maximum(r0, r1)
    rgt = mr + jnp.log(jnp.exp(r0 - mr) + jnp.exp(r1 - mr))

    g1s = _shift_k(g1)
    d0 = g1s[0] + _GO
    d1 = g1s[1] + _GO
    d2 = g1s[2] + _GE
    md = jnp.maximum(jnp.maximum(d0, d1), d2)
    dwn = md + jnp.log(jnp.exp(d0 - md) + jnp.exp(d1 - md)
                       + jnp.exp(d2 - md))

    g0 = jnp.stack([aln, rgt, dwn])

    v = g0 + z
    vmax = jnp.max(v, axis=0, keepdims=True)
    m_old = m_ref[...]
    m_new = jnp.maximum(m_old, vmax)
    s_ref[...] = (s_ref[...] * jnp.exp(m_old - m_new)
                  + jnp.sum(jnp.exp(v - m_new), axis=0, keepdims=True))
    m_ref[...] = m_new

    g2_ref[...] = g1
    g1_ref[...] = g0

    @pl.when(t == _N - 1)
    def _():
        mm = m_ref[...]
        gm = jnp.max(mm, keepdims=True)
        ssum = jnp.sum(s_ref[...] * jnp.exp(mm - gm), keepdims=True)
        out_ref[...] = (gm + jnp.log(ssum)).reshape(1, 1)


@jax.jit
def _sw_loss(x):
    xc = x[:-1, :-1]
    x2 = x[1:, 1:]
    yt = jnp.pad(_skew(xc).T, ((0, 0), (0, _KP - _A)),
                 constant_values=_NEG).reshape(_N, 1, _KP)
    zt = jnp.pad(_skew(x2).T, ((0, 0), (0, _KP - _A)),
                 constant_values=_NEG).reshape(_N, 1, _KP)

    out = pl.pallas_call(
        _sw_kernel,
        grid=(_N,),
        in_specs=[
            pl.BlockSpec((1, 1, _KP), lambda i: (i, 0, 0)),
            pl.BlockSpec((1, 1, _KP), lambda i: (i, 0, 0)),
        ],
        out_specs=pl.BlockSpec((1, 1), lambda i: (0, 0)),
        out_shape=jax.ShapeDtypeStruct((1, 1), jnp.float32),
        scratch_shapes=[
            pltpu.VMEM((3, _KP), jnp.float32),
            pltpu.VMEM((3, _KP), jnp.float32),
            pltpu.VMEM((1, _KP), jnp.float32),
            pltpu.VMEM((1, _KP), jnp.float32),
        ],
        compiler_params=pltpu.CompilerParams(
            dimension_semantics=("arbitrary",)),
    )(yt, zt)
    return out[0, 0]


def kernel(x):
    return _sw_loss(x)


# speedup vs baseline: 30.9759x; 3.0521x over previous
import functools

import jax
import jax.numpy as jnp
from jax.experimental import pallas as pl
from jax.experimental.pallas import tpu as pltpu

_GO = -11.0
_GE = -1.0
_NEG = -1e30
_A = 1023
_N = 2 * _A - 1
_KP = 1024
_U = 8
_NPAD = 2048


def _skew(mat):
    a, b = mat.shape
    n = a + b - 1
    p = jnp.pad(mat, ((0, 0), (0, n + 1 - b)), constant_values=_NEG)
    return p.reshape(-1)[: a * n].reshape(a, n)


def _shift_k(u):
    return jnp.concatenate(
        [jnp.full(u.shape[:-1] + (1,), _NEG, u.dtype), u[..., :-1]], axis=-1)


def _sw_kernel(y_ref, z_ref, out_ref, g2_ref, g1_ref, m_ref, s_ref):
    t = pl.program_id(0)

    @pl.when(t == 0)
    def _():
        g2_ref[...] = jnp.full((3, _KP), _NEG, jnp.float32)
        g1_ref[...] = jnp.full((3, _KP), _NEG, jnp.float32)
        m_ref[...] = jnp.full((1, _KP), _NEG, jnp.float32)
        s_ref[...] = jnp.zeros((1, _KP), jnp.float32)

    g2 = g2_ref[...]
    g1 = g1_ref[...]
    m_run = m_ref[0, :]
    s_run = s_ref[0, :]

    for u in range(_U):
        y = y_ref[u, 0, :]
        z = z_ref[u, 0, :]

        g2s = _shift_k(g2)
        m0 = jnp.maximum(jnp.maximum(g2s[0], g2s[1]),
                         jnp.maximum(g2s[2], 0.0))
        aln = y + m0 + jnp.log(
            jnp.exp(g2s[0] - m0) + jnp.exp(g2s[1] - m0)
            + jnp.exp(g2s[2] - m0) + jnp.exp(-m0))

        r0 = g1[0] + _GO
        r1 = g1[1] + _GE
        mr = jnp.maximum(r0, r1)
        rgt = mr + jnp.log(jnp.exp(r0 - mr) + jnp.exp(r1 - mr))

        g1s = _shift_k(g1)
        d0 = g1s[0] + _GO
        d1 = g1s[1] + _GO
        d2 = g1s[2] + _GE
        md = jnp.maximum(jnp.maximum(d0, d1), d2)
        dwn = md + jnp.log(jnp.exp(d0 - md) + jnp.exp(d1 - md)
                           + jnp.exp(d2 - md))

        g0 = jnp.stack([aln, rgt, dwn])

        v0 = aln + z
        v1 = rgt + z
        v2 = dwn + z
        vmax = jnp.maximum(jnp.maximum(v0, v1), v2)
        m_new = jnp.maximum(m_run, vmax)
        s_run = (s_run * jnp.exp(m_run - m_new)
                 + jnp.exp(v0 - m_new) + jnp.exp(v1 - m_new)
                 + jnp.exp(v2 - m_new))
        m_run = m_new

        g2 = g1
        g1 = g0

    g2_ref[...] = g2
    g1_ref[...] = g1
    m_ref[0, :] = m_run
    s_ref[0, :] = s_run

    @pl.when(t == _NPAD // _U - 1)
    def _():
        mm = m_ref[...]
        gm = jnp.max(mm, keepdims=True)
        ssum = jnp.sum(s_ref[...] * jnp.exp(mm - gm), keepdims=True)
        out_ref[...] = (gm + jnp.log(ssum)).reshape(1, 1)


@jax.jit
def _sw_loss(x):
    xc = x[:-1, :-1]
    x2 = x[1:, 1:]
    yt = jnp.pad(_skew(xc).T, ((0, _NPAD - _N), (0, _KP - _A)),
                 constant_values=_NEG).reshape(_NPAD, 1, _KP)
    zt = jnp.pad(_skew(x2).T, ((0, _NPAD - _N), (0, _KP - _A)),
                 constant_values=_NEG).reshape(_NPAD, 1, _KP)

    out = pl.pallas_call(
        _sw_kernel,
        grid=(_NPAD // _U,),
        in_specs=[
            pl.BlockSpec((_U, 1, _KP), lambda i: (i, 0, 0)),
            pl.BlockSpec((_U, 1, _KP), lambda i: (i, 0, 0)),
        ],
        out_specs=pl.BlockSpec((1, 1), lambda i: (0, 0)),
        out_shape=jax.ShapeDtypeStruct((1, 1), jnp.float32),
        scratch_shapes=[
            pltpu.VMEM((3, _KP), jnp.float32),
            pltpu.VMEM((3, _KP), jnp.float32),
            pltpu.VMEM((1, _KP), jnp.float32),
            pltpu.VMEM((1, _KP), jnp.float32),
        ],
        compiler_params=pltpu.CompilerParams(
            dimension_semantics=("arbitrary",)),
    )(yt, zt)
    return out[0, 0]


def kernel(x):
    return _sw_loss(x)


# speedup vs baseline: 36.6445x; 1.1830x over previous
import functools

import jax
import jax.numpy as jnp
from jax.experimental import pallas as pl
from jax.experimental.pallas import tpu as pltpu

_GO = -11.0
_GE = -1.0
_NEG = -1e30
_A = 1023
_N = 2 * _A - 1
_KP = 1024
_U = 8
_NPAD = 2048


def _skew(mat):
    a, b = mat.shape
    n = a + b - 1
    p = jnp.pad(mat, ((0, 0), (0, n + 1 - b)), constant_values=_NEG)
    return p.reshape(-1)[: a * n].reshape(a, n)


def _shift_k(u, fill):
    return jnp.concatenate(
        [jnp.full(u.shape[:-1] + (1,), fill, u.dtype), u[..., :-1]], axis=-1)


_CGO = 1.670170079024566e-05
_CGE = 0.36787944117144233


def _sw_kernel(y_ref, z_ref, out_ref,
               h2_ref, h1_ref, r1_ref, f1_ref, m_ref, s_ref):
    t = pl.program_id(0)

    @pl.when(t == 0)
    def _():
        h2_ref[...] = jnp.zeros((1, _KP), jnp.float32)
        h1_ref[...] = jnp.zeros((1, _KP), jnp.float32)
        r1_ref[...] = jnp.full((1, _KP), _NEG, jnp.float32)
        f1_ref[...] = jnp.full((1, _KP), _NEG, jnp.float32)
        m_ref[...] = jnp.full((1, _KP), _NEG, jnp.float32)
        s_ref[...] = jnp.zeros((1, _KP), jnp.float32)

    h2 = h2_ref[0, :]
    h1 = h1_ref[0, :]
    r1 = r1_ref[0, :]
    f1 = f1_ref[0, :]
    m_run = m_ref[0, :]
    s_run = s_ref[0, :]

    for u in range(_U):
        y = y_ref[u, 0, :]
        z = z_ref[u, 0, :]

        aln = y + _shift_k(h2, 0.0)
        rgt = r1
        dwn = _shift_k(f1, _NEG)

        q = jnp.maximum(jnp.maximum(aln, rgt), jnp.maximum(dwn, 0.0))
        ea = jnp.exp(aln - q)
        er = jnp.exp(rgt - q)
        ed = jnp.exp(dwn - q)
        e0 = jnp.exp(-q)
        se3 = ea + er + ed

        h0 = q + jnp.log(se3 + e0)
        r0 = q + jnp.log(ea * _CGO + er * _CGE)
        f0 = q + jnp.log((ea + er) * _CGO + ed * _CGE)

        qz = q + z
        keep = m_run >= qz
        e = jnp.exp(jnp.where(keep, qz - m_run, m_run - qz))
        s_run = jnp.where(keep, s_run + se3 * e, s_run * e + se3)
        m_run = jnp.maximum(m_run, qz)

        h2 = h1
        h1 = h0
        r1 = r0
        f1 = f0

    h2_ref[0, :] = h2
    h1_ref[0, :] = h1
    r1_ref[0, :] = r1
    f1_ref[0, :] = f1
    m_ref[0, :] = m_run
    s_ref[0, :] = s_run

    @pl.when(t == _NPAD // _U - 1)
    def _():
        mm = m_ref[...]
        gm = jnp.max(mm, keepdims=True)
        ssum = jnp.sum(s_ref[...] * jnp.exp(mm - gm), keepdims=True)
        out_ref[...] = (gm + jnp.log(ssum)).reshape(1, 1)


@jax.jit
def _sw_loss(x):
    xc = x[:-1, :-1]
    x2 = x[1:, 1:]
    yt = jnp.pad(_skew(xc).T, ((0, _NPAD - _N), (0, _KP - _A)),
                 constant_values=_NEG).reshape(_NPAD, 1, _KP)
    zt = jnp.pad(_skew(x2).T, ((0, _NPAD - _N), (0, _KP - _A)),
                 constant_values=_NEG).reshape(_NPAD, 1, _KP)

    out = pl.pallas_call(
        _sw_kernel,
        grid=(_NPAD // _U,),
        in_specs=[
            pl.BlockSpec((_U, 1, _KP), lambda i: (i, 0, 0)),
            pl.BlockSpec((_U, 1, _KP), lambda i: (i, 0, 0)),
        ],
        out_specs=pl.BlockSpec((1, 1), lambda i: (0, 0)),
        out_shape=jax.ShapeDtypeStruct((1, 1), jnp.float32),
        scratch_shapes=[
            pltpu.VMEM((1, _KP), jnp.float32),
            pltpu.VMEM((1, _KP), jnp.float32),
            pltpu.VMEM((1, _KP), jnp.float32),
            pltpu.VMEM((1, _KP), jnp.float32),
            pltpu.VMEM((1, _KP), jnp.float32),
            pltpu.VMEM((1, _KP), jnp.float32),
        ],
        compiler_params=pltpu.CompilerParams(
            dimension_semantics=("arbitrary",)),
    )(yt, zt)
    return out[0, 0]


def kernel(x):
    return _sw_loss(x)
